# baseline (speedup 1.0000x reference)
import jax
import jax.numpy as jnp
from jax import lax
from jax.experimental import pallas as pl
from jax.experimental.pallas import tpu as pltpu
from jax.experimental.pallas import tpu_sc as plsc

N_NODES = 100000
N_EDGES = 6400000
NPAD = 100352

NC = 2
NS = 16
L = 16
NW = NC * NS
EPW = N_EDGES // NW
CHA = 4000
NCHA = EPW // CHA
CHC = 2000
NCHC = EPW // CHC
NB_T = 25
NPT = N_NODES // NB_T
BLKB = NPAD // 7

_MESH = dict(core_axis_name="c", subcore_axis_name="s", num_cores=NC,
             num_subcores=NS)
_PARAMS = dict(
    mesh=plsc.VectorSubcoreMesh(**_MESH),
    compiler_params=pltpu.CompilerParams(needs_layout_passes=False),
)


def _wid():
    return lax.axis_index("s") * NC + lax.axis_index("c")


def _deg_body(row_hbm, ew_hbm, part_hbm,
              rb0, rb1, wb0, wb1, deg, sr0, sr1, sw0, sw1):
    wid = _wid()
    ebase = wid * EPW
    bufs = ((rb0, wb0, sr0, sw0), (rb1, wb1, sr1, sw1))
    zeros16 = jnp.zeros((L,), jnp.float32)

    def zinit(i, _):
        deg[pl.ds(i * L, L)] = zeros16
        return 0
    lax.fori_loop(0, NPAD // L, zinit, 0, unroll=8)

    def issue(b, c):
        base = ebase + c * CHA
        rb, wb, sr, sw = bufs[b]
        pltpu.async_copy(row_hbm.at[pl.ds(base, CHA)], rb, sr)
        pltpu.async_copy(ew_hbm.at[pl.ds(base, CHA)], wb, sw)

    issue(0, 0)
    issue(1, 1)

    def outer(g, _):
        for b in range(2):
            cg = g * 2 + b
            rb, wb, sr, sw = bufs[b]
            pltpu.make_async_copy(row_hbm.at[pl.ds(0, CHA)], rb, sr).wait()
            pltpu.make_async_copy(ew_hbm.at[pl.ds(0, CHA)], wb, sw).wait()

            def body(j, _):
                s = pl.ds(j * L, L)
                plsc.addupdate_scatter(deg, [rb[s]], wb[s])
                return 0
            lax.fori_loop(0, CHA // L, body, 0, unroll=4)

            @pl.when(cg + 2 < NCHA)
            def _():
                issue(b, cg + 2)
        return 0
    lax.fori_loop(0, NCHA // 2, outer, 0)

    pltpu.sync_copy(deg, part_hbm.at[pl.ds(wid * NPAD, NPAD)])


@jax.jit
def _deg_kernel(row, edge_weight):
    return pl.kernel(
        _deg_body,
        out_type=jax.ShapeDtypeStruct((NW * NPAD,), jnp.float32),
        scratch_types=[
            pltpu.VMEM((CHA,), jnp.int32),
            pltpu.VMEM((CHA,), jnp.int32),
            pltpu.VMEM((CHA,), jnp.float32),
            pltpu.VMEM((CHA,), jnp.float32),
            pltpu.VMEM((NPAD,), jnp.float32),
            pltpu.SemaphoreType.DMA,
            pltpu.SemaphoreType.DMA,
            pltpu.SemaphoreType.DMA,
            pltpu.SemaphoreType.DMA,
        ],
        **_PARAMS,
    )(row, edge_weight)


def _reduce_body(part_hbm, dis_ref, b0, b1, s0, s1):
    i = pl.program_id(0)
    base = i * BLKB
    bufs = ((b0, s0), (b1, s1))

    def issue(b, k):
        buf, sem = bufs[b]
        pltpu.make_async_copy(
            part_hbm.at[pl.ds(k * NPAD + base, BLKB)], buf, sem).start()

    issue(0, 0)
    issue(1, 1)

    def outer(g, acc):
        for b in range(2):
            k = g * 2 + b
            buf, sem = bufs[b]
            pltpu.make_async_copy(
                part_hbm.at[pl.ds(0, BLKB)], buf, sem).wait()
            acc = acc + buf[...]

            @pl.when(k + 2 < NW)
            def _():
                issue(b, k + 2)
        return acc

    acc = lax.fori_loop(0, NW // 2, outer, jnp.ones((BLKB,), jnp.float32))
    dis_ref[...] = lax.rsqrt(acc)


@jax.jit
def _reduce_kernel(part):
    return pl.pallas_call(
        _reduce_body,
        grid=(NPAD // BLKB,),
        in_specs=[pl.BlockSpec(memory_space=pl.ANY)],
        out_specs=pl.BlockSpec((BLKB,), lambda i: (i,)),
        out_shape=jax.ShapeDtypeStruct((NPAD,), jnp.float32),
        scratch_shapes=[
            pltpu.VMEM((BLKB,), jnp.float32),
            pltpu.VMEM((BLKB,), jnp.float32),
            pltpu.SemaphoreType.DMA,
            pltpu.SemaphoreType.DMA,
        ],
    )(part)


def _norm_body(row_hbm, col_hbm, ew_hbm, dis_hbm, out_hbm, disb, tbuf,
               rb0, rb1, cb0, cb1, wb0, wb1, ob0, ob1,
               sd, st, sr0, sr1, sc0, sc1, sw0, sw1, so0, so1):
    wid = _wid()
    ebase = wid * EPW
    bufs = ((rb0, cb0, wb0, ob0, sr0, sc0, sw0, so0),
            (rb1, cb1, wb1, ob1, sr1, sc1, sw1, so1))

    cpdis = pltpu.async_copy(dis_hbm, disb, sd)

    def issue(b, c):
        base = ebase + c * CHC
        rb, cb, wb = bufs[b][0], bufs[b][1], bufs[b][2]
        sr, sc, sw = bufs[b][4], bufs[b][5], bufs[b][6]
        pltpu.async_copy(row_hbm.at[pl.ds(base, CHC)], rb, sr)
        pltpu.async_copy(col_hbm.at[pl.ds(base, CHC)], cb, sc)
        pltpu.async_copy(ew_hbm.at[pl.ds(base, CHC)], wb, sw)

    issue(0, 0)
    issue(1, 1)
    cpdis.wait()

    @pl.when(wid < NB_T)
    def _():
        def sbody(i, _):
            v = disb[pl.ds(wid * NPT + i * L, L)]
            tbuf[pl.ds(i * L, L)] = v * v
            return 0
        lax.fori_loop(0, NPT // L, sbody, 0, unroll=4)
        pltpu.async_copy(
            tbuf, out_hbm.at[pl.ds(N_EDGES + wid * NPT, NPT)], st)

    def outer(g, _):
        for b in range(2):
            cg = g * 2 + b
            rb, cb, wb, ob, sr, sc, sw, so = bufs[b]
            pltpu.make_async_copy(row_hbm.at[pl.ds(0, CHC)], rb, sr).wait()
            pltpu.make_async_copy(col_hbm.at[pl.ds(0, CHC)], cb, sc).wait()
            pltpu.make_async_copy(ew_hbm.at[pl.ds(0, CHC)], wb, sw).wait()

            @pl.when(cg >= 2)
            def _():
                pltpu.make_async_copy(
                    ob, out_hbm.at[pl.ds(0, CHC)], so).wait()

            def body(j, _):
                s = pl.ds(j * L, L)
                dr = plsc.load_gather(disb, [rb[s]])
                dc = plsc.load_gather(disb, [cb[s]])
                ob[s] = dr * wb[s] * dc
                return 0
            lax.fori_loop(0, CHC // L, body, 0, unroll=4)

            pltpu.async_copy(ob, out_hbm.at[pl.ds(ebase + cg * CHC, CHC)], so)

            @pl.when(cg + 2 < NCHC)
            def _():
                issue(b, cg + 2)
        return 0
    lax.fori_loop(0, NCHC // 2, outer, 0)

    for b in range(2):
        ob, so = bufs[b][3], bufs[b][7]
        pltpu.make_async_copy(ob, out_hbm.at[pl.ds(0, CHC)], so).wait()

    @pl.when(wid < NB_T)
    def _():
        pltpu.make_async_copy(
            tbuf, out_hbm.at[pl.ds(0, NPT)], st).wait()


@jax.jit
def _norm_kernel(row, col, edge_weight, dis):
    return pl.kernel(
        _norm_body,
        out_type=jax.ShapeDtypeStruct((N_EDGES + N_NODES,), jnp.float32),
        scratch_types=[
            pltpu.VMEM((NPAD,), jnp.float32),
            pltpu.VMEM((NPT,), jnp.float32),
            pltpu.VMEM((CHC,), jnp.int32),
            pltpu.VMEM((CHC,), jnp.int32),
            pltpu.VMEM((CHC,), jnp.int32),
            pltpu.VMEM((CHC,), jnp.int32),
            pltpu.VMEM((CHC,), jnp.float32),
            pltpu.VMEM((CHC,), jnp.float32),
            pltpu.VMEM((CHC,), jnp.float32),
            pltpu.VMEM((CHC,), jnp.float32),
            pltpu.SemaphoreType.DMA,
            pltpu.SemaphoreType.DMA,
            pltpu.SemaphoreType.DMA,
            pltpu.SemaphoreType.DMA,
            pltpu.SemaphoreType.DMA,
            pltpu.SemaphoreType.DMA,
            pltpu.SemaphoreType.DMA,
            pltpu.SemaphoreType.DMA,
            pltpu.SemaphoreType.DMA,
            pltpu.SemaphoreType.DMA,
        ],
        **_PARAMS,
    )(row, col, edge_weight, dis)


def kernel(edge_index, edge_weight):
    row = edge_index[:, 0]
    col = edge_index[:, 1]
    diag = jnp.arange(N_NODES, dtype=edge_index.dtype)
    ei = jnp.concatenate(
        [edge_index, jnp.stack([diag, diag], axis=1)], axis=0)
    part = _deg_kernel(row, edge_weight)
    dis = _reduce_kernel(part)
    normed = _norm_kernel(row, col, edge_weight, dis)
    return (ei, normed)


# speedup vs baseline: 1.1724x; 1.1724x over previous
import jax
import jax.numpy as jnp
from jax import lax
from jax.experimental import pallas as pl
from jax.experimental.pallas import tpu as pltpu
from jax.experimental.pallas import tpu_sc as plsc

N_NODES = 100000
N_EDGES = 6400000
NPAD = 100352

NC = 2
NS = 16
L = 16
NW = NC * NS
EPW = N_EDGES // NW
CHA = 4000
NCHA = EPW // CHA
CHC = 2000
NCHC = EPW // CHC
NB_T = 25
NPT = N_NODES // NB_T

_MESH = dict(core_axis_name="c", subcore_axis_name="s", num_cores=NC,
             num_subcores=NS)
_PARAMS = dict(
    mesh=plsc.VectorSubcoreMesh(**_MESH),
    compiler_params=pltpu.CompilerParams(needs_layout_passes=False),
)


def _wid():
    return lax.axis_index("s") * NC + lax.axis_index("c")


def _deg_body(row_hbm, ew_hbm, part_hbm,
              rb0, rb1, wb0, wb1, deg, sr0, sr1, sw0, sw1):
    wid = _wid()
    ebase = wid * EPW
    bufs = ((rb0, wb0, sr0, sw0), (rb1, wb1, sr1, sw1))
    zeros16 = jnp.zeros((L,), jnp.float32)

    def zinit(i, _):
        deg[pl.ds(i * L, L)] = zeros16
        return 0
    lax.fori_loop(0, NPAD // L, zinit, 0, unroll=8)

    def issue(b, c):
        base = ebase + c * CHA
        rb, wb, sr, sw = bufs[b]
        pltpu.async_copy(row_hbm.at[pl.ds(base, CHA)], rb, sr)
        pltpu.async_copy(ew_hbm.at[pl.ds(base, CHA)], wb, sw)

    issue(0, 0)
    issue(1, 1)

    def outer(g, _):
        for b in range(2):
            cg = g * 2 + b
            rb, wb, sr, sw = bufs[b]
            pltpu.make_async_copy(row_hbm.at[pl.ds(0, CHA)], rb, sr).wait()
            pltpu.make_async_copy(ew_hbm.at[pl.ds(0, CHA)], wb, sw).wait()

            def body(j, _):
                s = pl.ds(j * L, L)
                plsc.addupdate_scatter(deg, [rb[s]], wb[s])
                return 0
            lax.fori_loop(0, CHA // L, body, 0, unroll=4)

            @pl.when(cg + 2 < NCHA)
            def _():
                issue(b, cg + 2)
        return 0
    lax.fori_loop(0, NCHA // 2, outer, 0)

    pltpu.sync_copy(deg, part_hbm.at[pl.ds(wid * NPAD, NPAD)])


@jax.jit
def _deg_kernel(row, edge_weight):
    return pl.kernel(
        _deg_body,
        out_type=jax.ShapeDtypeStruct((NW * NPAD,), jnp.float32),
        scratch_types=[
            pltpu.VMEM((CHA,), jnp.int32),
            pltpu.VMEM((CHA,), jnp.int32),
            pltpu.VMEM((CHA,), jnp.float32),
            pltpu.VMEM((CHA,), jnp.float32),
            pltpu.VMEM((NPAD,), jnp.float32),
            pltpu.SemaphoreType.DMA,
            pltpu.SemaphoreType.DMA,
            pltpu.SemaphoreType.DMA,
            pltpu.SemaphoreType.DMA,
        ],
        **_PARAMS,
    )(row, edge_weight)


_NRING = 4
NPB = NPAD // NW


def _rsqrt16(x):
    xi = plsc.bitcast(x, jnp.int32)
    yi = jnp.full((L,), 0x5F3759DF, jnp.int32) - lax.shift_right_logical(
        xi, jnp.full((L,), 1, jnp.int32))
    y = plsc.bitcast(yi, jnp.float32)
    half = jnp.full((L,), 0.5, jnp.float32)
    three_half = jnp.full((L,), 1.5, jnp.float32)
    for _ in range(3):
        y = y * (three_half - half * x * y * y)
    return y


def _reduce_body(part_hbm, dis_hbm, acc, b0, b1, b2, b3, s0, s1, s2, s3):
    wid = _wid()
    bufs = ((b0, s0), (b1, s1), (b2, s2), (b3, s3))
    base = wid * NPB
    ones16 = jnp.full((L,), 1.0, jnp.float32)

    def issue(r, k):
        buf, sem = bufs[r]
        pltpu.async_copy(
            part_hbm.at[pl.ds(k * NPAD + base, NPB)], buf, sem)

    for r in range(_NRING):
        issue(r, r)

    def init(i, _):
        acc[pl.ds(i * L, L)] = ones16
        return 0
    lax.fori_loop(0, NPB // L, init, 0, unroll=8)

    def outer(g, _):
        for r in range(_NRING):
            k = g * _NRING + r
            buf, sem = bufs[r]
            pltpu.make_async_copy(
                part_hbm.at[pl.ds(0, NPB)], buf, sem).wait()

            def add(i, _):
                s = pl.ds(i * L, L)
                acc[s] = acc[s] + buf[s]
                return 0
            lax.fori_loop(0, NPB // L, add, 0, unroll=8)

            @pl.when(k + _NRING < NW)
            def _():
                issue(r, k + _NRING)
        return 0
    lax.fori_loop(0, NW // _NRING, outer, 0)

    def finish(i, _):
        s = pl.ds(i * L, L)
        acc[s] = _rsqrt16(acc[s])
        return 0
    lax.fori_loop(0, NPB // L, finish, 0, unroll=4)

    pltpu.sync_copy(acc, dis_hbm.at[pl.ds(base, NPB)])


@jax.jit
def _reduce_kernel(part):
    return pl.kernel(
        _reduce_body,
        out_type=jax.ShapeDtypeStruct((NPAD,), jnp.float32),
        scratch_types=[
            pltpu.VMEM((NPB,), jnp.float32),
            pltpu.VMEM((NPB,), jnp.float32),
            pltpu.VMEM((NPB,), jnp.float32),
            pltpu.VMEM((NPB,), jnp.float32),
            pltpu.VMEM((NPB,), jnp.float32),
            pltpu.SemaphoreType.DMA,
            pltpu.SemaphoreType.DMA,
            pltpu.SemaphoreType.DMA,
            pltpu.SemaphoreType.DMA,
        ],
        **_PARAMS,
    )(part)


def _norm_body(row_hbm, col_hbm, ew_hbm, dis_hbm, out_hbm, disb, tbuf,
               rb0, rb1, cb0, cb1, wb0, wb1, ob0, ob1,
               sd, st, sr0, sr1, sc0, sc1, sw0, sw1, so0, so1):
    wid = _wid()
    ebase = wid * EPW
    bufs = ((rb0, cb0, wb0, ob0, sr0, sc0, sw0, so0),
            (rb1, cb1, wb1, ob1, sr1, sc1, sw1, so1))

    cpdis = pltpu.async_copy(dis_hbm, disb, sd)

    def issue(b, c):
        base = ebase + c * CHC
        rb, cb, wb = bufs[b][0], bufs[b][1], bufs[b][2]
        sr, sc, sw = bufs[b][4], bufs[b][5], bufs[b][6]
        pltpu.async_copy(row_hbm.at[pl.ds(base, CHC)], rb, sr)
        pltpu.async_copy(col_hbm.at[pl.ds(base, CHC)], cb, sc)
        pltpu.async_copy(ew_hbm.at[pl.ds(base, CHC)], wb, sw)

    issue(0, 0)
    issue(1, 1)
    cpdis.wait()

    @pl.when(wid < NB_T)
    def _():
        def sbody(i, _):
            v = disb[pl.ds(wid * NPT + i * L, L)]
            tbuf[pl.ds(i * L, L)] = v * v
            return 0
        lax.fori_loop(0, NPT // L, sbody, 0, unroll=4)
        pltpu.async_copy(
            tbuf, out_hbm.at[pl.ds(N_EDGES + wid * NPT, NPT)], st)

    def outer(g, _):
        for b in range(2):
            cg = g * 2 + b
            rb, cb, wb, ob, sr, sc, sw, so = bufs[b]
            pltpu.make_async_copy(row_hbm.at[pl.ds(0, CHC)], rb, sr).wait()
            pltpu.make_async_copy(col_hbm.at[pl.ds(0, CHC)], cb, sc).wait()
            pltpu.make_async_copy(ew_hbm.at[pl.ds(0, CHC)], wb, sw).wait()

            @pl.when(cg >= 2)
            def _():
                pltpu.make_async_copy(
                    ob, out_hbm.at[pl.ds(0, CHC)], so).wait()

            def body(j, _):
                s = pl.ds(j * L, L)
                dr = plsc.load_gather(disb, [rb[s]])
                dc = plsc.load_gather(disb, [cb[s]])
                ob[s] = dr * wb[s] * dc
                return 0
            lax.fori_loop(0, CHC // L, body, 0, unroll=4)

            pltpu.async_copy(ob, out_hbm.at[pl.ds(ebase + cg * CHC, CHC)], so)

            @pl.when(cg + 2 < NCHC)
            def _():
                issue(b, cg + 2)
        return 0
    lax.fori_loop(0, NCHC // 2, outer, 0)

    for b in range(2):
        ob, so = bufs[b][3], bufs[b][7]
        pltpu.make_async_copy(ob, out_hbm.at[pl.ds(0, CHC)], so).wait()

    @pl.when(wid < NB_T)
    def _():
        pltpu.make_async_copy(
            tbuf, out_hbm.at[pl.ds(0, NPT)], st).wait()


@jax.jit
def _norm_kernel(row, col, edge_weight, dis):
    return pl.kernel(
        _norm_body,
        out_type=jax.ShapeDtypeStruct((N_EDGES + N_NODES,), jnp.float32),
        scratch_types=[
            pltpu.VMEM((NPAD,), jnp.float32),
            pltpu.VMEM((NPT,), jnp.float32),
            pltpu.VMEM((CHC,), jnp.int32),
            pltpu.VMEM((CHC,), jnp.int32),
            pltpu.VMEM((CHC,), jnp.int32),
            pltpu.VMEM((CHC,), jnp.int32),
            pltpu.VMEM((CHC,), jnp.float32),
            pltpu.VMEM((CHC,), jnp.float32),
            pltpu.VMEM((CHC,), jnp.float32),
            pltpu.VMEM((CHC,), jnp.float32),
            pltpu.SemaphoreType.DMA,
            pltpu.SemaphoreType.DMA,
            pltpu.SemaphoreType.DMA,
            pltpu.SemaphoreType.DMA,
            pltpu.SemaphoreType.DMA,
            pltpu.SemaphoreType.DMA,
            pltpu.SemaphoreType.DMA,
            pltpu.SemaphoreType.DMA,
            pltpu.SemaphoreType.DMA,
            pltpu.SemaphoreType.DMA,
        ],
        **_PARAMS,
    )(row, col, edge_weight, dis)


def kernel(edge_index, edge_weight):
    row = edge_index[:, 0]
    col = edge_index[:, 1]
    diag = jnp.arange(N_NODES, dtype=edge_index.dtype)
    ei = jnp.concatenate(
        [edge_index, jnp.stack([diag, diag], axis=1)], axis=0)
    part = _deg_kernel(row, edge_weight)
    dis = _reduce_kernel(part)
    normed = _norm_kernel(row, col, edge_weight, dis)
    return (ei, normed)
